# baseline (speedup 1.0000x reference)
import jax
import jax.numpy as jnp
from jax import lax
from jax.experimental import pallas as pl
from jax.experimental.pallas import tpu as pltpu

_B, _S, _V = 16, 128, 5000
_N = 4
_NVB = 5
_VB = _V // _NVB
_NCB = _VB // 8
_NBANK = 5
_PER = _NCB // _NBANK
_KEY_HI, _KEY_LO = 0, 42


def _threefry2x32(x0, x1):
    k0 = jnp.uint32(_KEY_HI)
    k1 = jnp.uint32(_KEY_LO)
    ks2 = jnp.uint32(0x1BD11BDA) ^ k0 ^ k1
    ks = (k0, k1, ks2)
    rots = ((13, 15, 26, 6), (17, 29, 16, 24))
    x0 = x0 + k0
    x1 = x1 + k1
    for i in range(5):
        for r in rots[i % 2]:
            x0 = x0 + x1
            x1 = (x1 << r) | (x1 >> (32 - r))
            x1 = x1 ^ x0
        x0 = x0 + ks[(i + 1) % 3]
        x1 = x1 + ks[(i + 2) % 3] + jnp.uint32(i + 1)
    return x0, x1


def _mwer_body(len_ref, xt_ref, tgt_ref, out_ref, m1s, i1s_, m2s, i2s_):
    b = pl.program_id(0)
    v = pl.program_id(1)
    neg = jnp.full((8, _S), -jnp.inf, jnp.float32)
    zero = jnp.zeros((8, _S), jnp.int32)
    big = _V

    def step(j, carry):
        new = []
        for k in range(_NBANK):
            a1, p1, a2, p2 = carry[4 * k:4 * k + 4]
            cj = k * _PER + j
            c = xt_ref[0, pl.ds(cj * 8, 8), :]
            bj = jnp.full((8, _S), cj, jnp.int32)
            gt1 = c > a1
            gt2 = c > a2
            new.append(jnp.where(gt1, c, a1))
            new.append(jnp.where(gt1, bj, p1))
            new.append(jnp.where(gt1, a1, jnp.where(gt2, c, a2)))
            new.append(jnp.where(gt1, p1, jnp.where(gt2, bj, p2)))
        return tuple(new)

    init = (neg, zero, neg, zero) * _NBANK
    res = lax.fori_loop(0, _PER, step, init, unroll=5)

    row8 = lax.broadcasted_iota(jnp.int32, (8, _S), 0)
    voff = v * _VB
    a1 = jnp.concatenate([res[4 * k] for k in range(_NBANK)], axis=0)
    i1c = jnp.concatenate(
        [res[4 * k + 1] * 8 + row8 + voff for k in range(_NBANK)], axis=0)
    a2 = jnp.concatenate([res[4 * k + 2] for k in range(_NBANK)], axis=0)
    i2c = jnp.concatenate(
        [res[4 * k + 3] * 8 + row8 + voff for k in range(_NBANK)], axis=0)

    m1 = jnp.max(a1, axis=0, keepdims=True)
    i1 = jnp.min(jnp.where(a1 == m1, i1c, big), axis=0, keepdims=True)
    win = jnp.logical_and(a1 == m1, i1c == i1)
    c2 = jnp.where(win, a2, a1)
    c2i = jnp.where(win, i2c, i1c)
    m2 = jnp.max(c2, axis=0, keepdims=True)
    i2 = jnp.min(jnp.where(c2 == m2, c2i, big), axis=0, keepdims=True)

    @pl.when(v == 0)
    def _():
        m1s[...] = m1
        i1s_[...] = i1
        m2s[...] = m2
        i2s_[...] = i2

    @pl.when(v > 0)
    def _():
        pm1 = m1s[...]
        pi1 = i1s_[...]
        pm2 = m2s[...]
        pi2 = i2s_[...]
        gt = m1 > pm1
        rm_a = jnp.where(m2 > pm1, m2, pm1)
        ri_a = jnp.where(m2 > pm1, i2, pi1)
        rm_b = jnp.where(m1 > pm2, m1, pm2)
        ri_b = jnp.where(m1 > pm2, i1, pi2)
        m1s[...] = jnp.where(gt, m1, pm1)
        i1s_[...] = jnp.where(gt, i1, pi1)
        m2s[...] = jnp.where(gt, rm_a, rm_b)
        i2s_[...] = jnp.where(gt, ri_a, ri_b)

    @pl.when(v == _NVB - 1)
    def _():
        m1f = m1s[...]
        i1f = i1s_[...]
        m2f = m2s[...]
        i2f = i2s_[...]

        n_iota = lax.broadcasted_iota(jnp.int32, (_N, _S), 0)
        s_iota = lax.broadcasted_iota(jnp.int32, (_N, _S), 1)
        i1r = jnp.broadcast_to(i1f, (_N, _S))
        flat = ((n_iota * _B + b) * _S + s_iota) * _V + i1r
        o0, o1 = _threefry2x32(jnp.zeros((_N, _S), jnp.uint32),
                               flat.astype(jnp.uint32))
        bits = o0 ^ o1
        masked = (bits >> 31) == 0

        pad = s_iota >= len_ref[b]
        v1 = jnp.broadcast_to(m1f, (_N, _S))
        v2 = jnp.broadcast_to(m2f, (_N, _S))
        i2r = jnp.broadcast_to(i2f, (_N, _S))

        sel_v = jnp.where(masked, v2, v1)
        sel_v = jnp.where(pad, 0.0, sel_v)
        a = jnp.sum(sel_v, axis=-1, keepdims=True)

        pred = jnp.where(masked, i2r, i1r)
        tgt = jnp.broadcast_to(tgt_ref[0, 0].reshape(1, _S), (_N, _S))
        err = jnp.sum(
            jnp.where(pad, 0.0, (tgt != pred).astype(jnp.float32)),
            axis=-1, keepdims=True)

        md = jnp.max(a, axis=0, keepdims=True)
        w = jnp.exp(a - md)
        normal = w / jnp.sum(w, axis=0, keepdims=True)
        dev = err - jnp.mean(err, axis=0, keepdims=True)
        loss_b = jnp.sum(normal * dev, axis=0, keepdims=True) * (1.0 / _B)

        @pl.when(b == 0)
        def _():
            out_ref[0] = jnp.zeros((1, 1), jnp.float32)

        out_ref[0] += loss_b


def kernel(logit, tgt, tgt_lens):
    xt = jnp.transpose(logit, (0, 2, 1))
    tgt3 = tgt.reshape(_B, 1, _S)
    loss = pl.pallas_call(
        _mwer_body,
        grid=(_B, _NVB),
        in_specs=[
            pl.BlockSpec(memory_space=pltpu.SMEM),
            pl.BlockSpec((1, _VB, _S), lambda b, v: (b, v, 0)),
            pl.BlockSpec((1, 1, _S), lambda b, v: (b, 0, 0)),
        ],
        out_specs=pl.BlockSpec((1, 1, 1), lambda b, v: (0, 0, 0)),
        out_shape=jax.ShapeDtypeStruct((1, 1, 1), jnp.float32),
        scratch_shapes=[
            pltpu.VMEM((1, _S), jnp.float32),
            pltpu.VMEM((1, _S), jnp.int32),
            pltpu.VMEM((1, _S), jnp.float32),
            pltpu.VMEM((1, _S), jnp.int32),
        ],
    )(tgt_lens, xt, tgt3)
    return loss[0, 0, 0]


# speedup vs baseline: 1.8603x; 1.8603x over previous
import jax
import jax.numpy as jnp
from jax import lax
from jax.experimental import pallas as pl
from jax.experimental.pallas import tpu as pltpu

_B, _S, _V = 16, 128, 5000
_N = 4
_NQ = 5
_VB = _V // _NQ
_NCB = _VB // 8
_NBANK = 5
_PER = _NCB // _NBANK
_NSLOT = 4
_AHEAD = 3
_KEY_HI, _KEY_LO = 0, 42


def _threefry2x32(x0, x1):
    k0 = jnp.uint32(_KEY_HI)
    k1 = jnp.uint32(_KEY_LO)
    ks2 = jnp.uint32(0x1BD11BDA) ^ k0 ^ k1
    ks = (k0, k1, ks2)
    rots = ((13, 15, 26, 6), (17, 29, 16, 24))
    x0 = x0 + k0
    x1 = x1 + k1
    for i in range(5):
        for r in rots[i % 2]:
            x0 = x0 + x1
            x1 = (x1 << r) | (x1 >> (32 - r))
            x1 = x1 ^ x0
        x0 = x0 + ks[(i + 1) % 3]
        x1 = x1 + ks[(i + 2) % 3] + jnp.uint32(i + 1)
    return x0, x1


def _mwer_body(len_ref, tgt_ref, xt_hbm, out_ref, xbuf, sem):
    b = pl.program_id(0)
    neg = jnp.full((8, _S), -jnp.inf, jnp.float32)
    zero = jnp.zeros((8, _S), jnp.int32)
    big = _V

    def dma(bb, q, slot):
        return pltpu.make_async_copy(
            xt_hbm.at[bb, pl.ds(q * _VB, _VB)], xbuf.at[slot], sem.at[slot])

    @pl.when(b == 0)
    def _():
        for q in range(_AHEAD):
            dma(0, q, q).start()

    carry = (neg, zero, neg, zero) * _NBANK
    for q in range(_NQ):
        g = b * _NQ + q
        slot = lax.rem(g, _NSLOT)

        nb, nq = divmod(q + _AHEAD, _NQ)

        @pl.when(b + nb < _B)
        def _(nb=nb, nq=nq, g=g):
            dma(b + nb, nq, lax.rem(g + _AHEAD, _NSLOT)).start()

        dma(b, q, slot).wait()

        def step(j, carry, _q=q, _slot=slot):
            new = []
            for k in range(_NBANK):
                a1, p1, a2, p2 = carry[4 * k:4 * k + 4]
                cj = k * _PER + j
                c = xbuf[_slot, pl.ds(cj * 8, 8), :]
                bj = jnp.full((8, _S), _q * _NCB + cj, jnp.int32)
                gt1 = c > a1
                gt2 = c > a2
                new.append(jnp.where(gt1, c, a1))
                new.append(jnp.where(gt1, bj, p1))
                new.append(jnp.where(gt1, a1, jnp.where(gt2, c, a2)))
                new.append(jnp.where(gt1, p1, jnp.where(gt2, bj, p2)))
            return tuple(new)

        carry = lax.fori_loop(0, _PER, step, carry, unroll=5)

    res = carry

    row8 = lax.broadcasted_iota(jnp.int32, (8, _S), 0)
    a1 = jnp.concatenate([res[4 * k] for k in range(_NBANK)], axis=0)
    i1c = jnp.concatenate(
        [res[4 * k + 1] * 8 + row8 for k in range(_NBANK)], axis=0)
    a2 = jnp.concatenate([res[4 * k + 2] for k in range(_NBANK)], axis=0)
    i2c = jnp.concatenate(
        [res[4 * k + 3] * 8 + row8 for k in range(_NBANK)], axis=0)

    m1 = jnp.max(a1, axis=0, keepdims=True)
    i1 = jnp.min(jnp.where(a1 == m1, i1c, big), axis=0, keepdims=True)
    win = jnp.logical_and(a1 == m1, i1c == i1)
    c2 = jnp.where(win, a2, a1)
    c2i = jnp.where(win, i2c, i1c)
    m2 = jnp.max(c2, axis=0, keepdims=True)
    i2 = jnp.min(jnp.where(c2 == m2, c2i, big), axis=0, keepdims=True)

    n_iota = lax.broadcasted_iota(jnp.int32, (_N, _S), 0)
    s_iota = lax.broadcasted_iota(jnp.int32, (_N, _S), 1)
    i1r = jnp.broadcast_to(i1, (_N, _S))
    flat = ((n_iota * _B + b) * _S + s_iota) * _V + i1r
    o0, o1 = _threefry2x32(jnp.zeros((_N, _S), jnp.uint32),
                           flat.astype(jnp.uint32))
    bits = o0 ^ o1
    masked = (bits >> 31) == 0

    pad = s_iota >= len_ref[b]
    v1 = jnp.broadcast_to(m1, (_N, _S))
    v2 = jnp.broadcast_to(m2, (_N, _S))
    i2r = jnp.broadcast_to(i2, (_N, _S))

    sel_v = jnp.where(masked, v2, v1)
    sel_v = jnp.where(pad, 0.0, sel_v)
    a = jnp.sum(sel_v, axis=-1, keepdims=True)

    pred = jnp.where(masked, i2r, i1r)
    tgt = jnp.broadcast_to(tgt_ref[0, 0].reshape(1, _S), (_N, _S))
    err = jnp.sum(
        jnp.where(pad, 0.0, (tgt != pred).astype(jnp.float32)),
        axis=-1, keepdims=True)

    md = jnp.max(a, axis=0, keepdims=True)
    w = jnp.exp(a - md)
    normal = w / jnp.sum(w, axis=0, keepdims=True)
    dev = err - jnp.mean(err, axis=0, keepdims=True)
    loss_b = jnp.sum(normal * dev, axis=0, keepdims=True) * (1.0 / _B)

    @pl.when(b == 0)
    def _():
        out_ref[0] = jnp.zeros((1, 1), jnp.float32)

    out_ref[0] += loss_b


def kernel(logit, tgt, tgt_lens):
    xt = jnp.transpose(logit, (0, 2, 1))
    tgt3 = tgt.reshape(_B, 1, _S)
    loss = pl.pallas_call(
        _mwer_body,
        grid=(_B,),
        in_specs=[
            pl.BlockSpec(memory_space=pltpu.SMEM),
            pl.BlockSpec((1, 1, _S), lambda b: (b, 0, 0)),
            pl.BlockSpec(memory_space=pl.ANY),
        ],
        out_specs=pl.BlockSpec((1, 1, 1), lambda b: (0, 0, 0)),
        out_shape=jax.ShapeDtypeStruct((1, 1, 1), jnp.float32),
        scratch_shapes=[
            pltpu.VMEM((_NSLOT, _VB, _S), jnp.float32),
            pltpu.SemaphoreType.DMA((_NSLOT,)),
        ],
    )(tgt_lens, tgt3, xt)
    return loss[0, 0, 0]


# speedup vs baseline: 2.3929x; 1.2863x over previous
import jax
import jax.numpy as jnp
from jax import lax
from jax.experimental import pallas as pl
from jax.experimental.pallas import tpu as pltpu

_B, _S, _V = 16, 128, 5000
_N = 4
_NC = _V // 8
_NBANK = 5
_KEY_HI, _KEY_LO = 0, 42


def _threefry2x32(x0, x1):
    k0 = jnp.uint32(_KEY_HI)
    k1 = jnp.uint32(_KEY_LO)
    ks2 = jnp.uint32(0x1BD11BDA) ^ k0 ^ k1
    ks = (k0, k1, ks2)
    rots = ((13, 15, 26, 6), (17, 29, 16, 24))
    x0 = x0 + k0
    x1 = x1 + k1
    for i in range(5):
        for r in rots[i % 2]:
            x0 = x0 + x1
            x1 = (x1 << r) | (x1 >> (32 - r))
            x1 = x1 ^ x0
        x0 = x0 + ks[(i + 1) % 3]
        x1 = x1 + ks[(i + 2) % 3] + jnp.uint32(i + 1)
    return x0, x1


def _mwer_body(len_ref, xt_ref, tgt_ref, out_ref):
    b = pl.program_id(0)
    neg = jnp.full((8, _S), -jnp.inf, jnp.float32)
    zero = jnp.zeros((8, _S), jnp.int32)

    nb = _NBANK
    per = _NC // nb

    def step(j, carry):
        new = []
        for k in range(nb):
            a1, p1, a2, p2 = carry[4 * k:4 * k + 4]
            cj = k * per + j
            c = xt_ref[0, pl.ds(cj * 8, 8), :]
            bj = jnp.full((8, _S), cj, jnp.int32)
            gt1 = c > a1
            gt2 = c > a2
            new.append(jnp.where(gt1, c, a1))
            new.append(jnp.where(gt1, bj, p1))
            new.append(jnp.where(gt1, a1, jnp.where(gt2, c, a2)))
            new.append(jnp.where(gt1, p1, jnp.where(gt2, bj, p2)))
        return tuple(new)

    init = (neg, zero, neg, zero) * nb
    res = lax.fori_loop(0, per, step, init, unroll=25)

    row8 = lax.broadcasted_iota(jnp.int32, (8, _S), 0)
    a1 = jnp.concatenate([res[4 * k] for k in range(nb)], axis=0)
    i1s = jnp.concatenate(
        [res[4 * k + 1] * 8 + row8 for k in range(nb)], axis=0)
    a2 = jnp.concatenate([res[4 * k + 2] for k in range(nb)], axis=0)
    i2s = jnp.concatenate(
        [res[4 * k + 3] * 8 + row8 for k in range(nb)], axis=0)

    big = _V
    m1 = jnp.max(a1, axis=0, keepdims=True)
    i1 = jnp.min(jnp.where(a1 == m1, i1s, big), axis=0, keepdims=True)
    win = jnp.logical_and(a1 == m1, i1s == i1)
    c2 = jnp.where(win, a2, a1)
    c2i = jnp.where(win, i2s, i1s)
    m2 = jnp.max(c2, axis=0, keepdims=True)
    i2 = jnp.min(jnp.where(c2 == m2, c2i, big), axis=0, keepdims=True)

    n_iota = lax.broadcasted_iota(jnp.int32, (_N, _S), 0)
    s_iota = lax.broadcasted_iota(jnp.int32, (_N, _S), 1)
    i1r = jnp.broadcast_to(i1, (_N, _S))
    flat = ((n_iota * _B + b) * _S + s_iota) * _V + i1r
    o0, o1 = _threefry2x32(jnp.zeros((_N, _S), jnp.uint32),
                           flat.astype(jnp.uint32))
    bits = o0 ^ o1
    masked = (bits >> 31) == 0

    pad = s_iota >= len_ref[b]
    v1 = jnp.broadcast_to(m1, (_N, _S))
    v2 = jnp.broadcast_to(m2, (_N, _S))
    i2r = jnp.broadcast_to(i2, (_N, _S))

    sel_v = jnp.where(masked, v2, v1)
    sel_v = jnp.where(pad, 0.0, sel_v)
    a = jnp.sum(sel_v, axis=-1, keepdims=True)

    pred = jnp.where(masked, i2r, i1r)
    tgt = jnp.broadcast_to(tgt_ref[0, 0].reshape(1, _S), (_N, _S))
    err = jnp.sum(
        jnp.where(pad, 0.0, (tgt != pred).astype(jnp.float32)),
        axis=-1, keepdims=True)

    md = jnp.max(a, axis=0, keepdims=True)
    w = jnp.exp(a - md)
    normal = w / jnp.sum(w, axis=0, keepdims=True)
    dev = err - jnp.mean(err, axis=0, keepdims=True)
    loss_b = jnp.sum(normal * dev, axis=0, keepdims=True) * (1.0 / _B)

    @pl.when(b == 0)
    def _():
        out_ref[0] = jnp.zeros((1, 1), jnp.float32)

    out_ref[0] += loss_b


def kernel(logit, tgt, tgt_lens):
    xt = jnp.transpose(logit, (0, 2, 1))
    tgt3 = tgt.reshape(_B, 1, _S)
    loss = pl.pallas_call(
        _mwer_body,
        grid=(_B,),
        in_specs=[
            pl.BlockSpec(memory_space=pltpu.SMEM),
            pl.BlockSpec((1, _V, _S), lambda b: (b, 0, 0)),
            pl.BlockSpec((1, 1, _S), lambda b: (b, 0, 0)),
        ],
        out_specs=pl.BlockSpec((1, 1, 1), lambda b: (0, 0, 0)),
        out_shape=jax.ShapeDtypeStruct((1, 1, 1), jnp.float32),
    )(tgt_lens, xt, tgt3)
    return loss[0, 0, 0]
